# baseline (speedup 1.0000x reference)
import functools

import jax
import jax.numpy as jnp
from jax import lax
from jax.experimental import pallas as pl
from jax.experimental.pallas import tpu as pltpu
from jax.experimental.pallas import tpu_sc as plsc

_N = 10000
_E = 320000
_DIN = 128
_DID = 16
_DH = 128
_DS = 136
_CB = 64
_NC = 2
_NS = 16
_NW = _NC * _NS
_NCHUNKS = _E // _CB
_NI_BASE = _NCHUNKS // _NW
_NI_EXTRA = _NCHUNKS - _NI_BASE * _NW
_NSLOT_QUADS = (_NI_BASE + 1 + 3) // 4
_NPAD = 10240
_ROWS_PER_TILE = _NPAD // _NS
_RC = 64


_BN = 1000


def _pre_body(x_ref, id_ref, wxa_ref, wxb_ref, wia_ref, wib_ref, b1_ref,
              p_ref, q_ref):
    xb = x_ref[...]
    idb = id_ref[...]
    p_ref[...] = (jnp.dot(xb, wxa_ref[...], preferred_element_type=jnp.float32)
                  + jnp.dot(idb, wia_ref[...], preferred_element_type=jnp.float32))
    q_ref[...] = (jnp.dot(xb, wxb_ref[...], preferred_element_type=jnp.float32)
                  + jnp.dot(idb, wib_ref[...], preferred_element_type=jnp.float32)
                  + b1_ref[...])


def _pre_call(x, ident, wxa, wxb, wia, wib, b1):
    grid = (_N // _BN,)
    full = lambda shape: pl.BlockSpec(shape, lambda i: (0, 0))
    return pl.pallas_call(
        _pre_body,
        grid=grid,
        in_specs=[
            pl.BlockSpec((_BN, _DIN), lambda i: (i, 0)),
            pl.BlockSpec((_BN, _DID), lambda i: (i, 0)),
            full((_DIN, _DH)),
            full((_DIN, _DH)),
            full((_DID, _DH)),
            full((_DID, _DH)),
            full((1, _DH)),
        ],
        out_specs=[
            pl.BlockSpec((_BN, _DH), lambda i: (i, 0)),
            pl.BlockSpec((_BN, _DH), lambda i: (i, 0)),
        ],
        out_shape=[
            jax.ShapeDtypeStruct((_N, _DH), jnp.float32),
            jax.ShapeDtypeStruct((_N, _DH), jnp.float32),
        ],
    )(x, ident, wxa, wxb, wia, wib, b1)


def _sc_edge_body(p_hbm, q_hbm, dst_hbm, src_hbm, ec_hbm, out_hbm,
                  dst_v, src_v, ec_v, p_v, q_v, h_v, s_sh,
                  sem_idx, sem_p, sem_q, sem_ec):
    cid = lax.axis_index("c")
    sid = lax.axis_index("s")
    wid = sid * _NC + cid
    n_i = _NI_BASE + jnp.where(wid < _NI_EXTRA, 1, 0)

    zero16 = jnp.zeros((16,), jnp.float32)

    def _zero_row(r, carry):
        for f in range(_DS // 16 + 1):
            off = min(16 * f, _DS - 16)
            h_v[r, pl.ds(off, 16)] = zero16
        return carry

    lax.fori_loop(0, _CB, _zero_row, 0)
    row0 = sid * _ROWS_PER_TILE
    for t in range(_ROWS_PER_TILE // _RC):
        pltpu.sync_copy(h_v.at[pl.ds(0, _RC)],
                        s_sh.at[pl.ds(row0 + t * _RC, _RC)])
    plsc.subcore_barrier()

    lane = lax.iota(jnp.int32, 16)
    lane_lt8 = lane < 8
    lane_eq8 = lane == 8
    idx_hi = ((lane % 8) + 8).reshape(16, 1)
    _dnums = lax.GatherDimensionNumbers(
        offset_dims=(), collapsed_slice_dims=(0,), start_index_map=(0,))

    def _vgather(vec, idx):
        return lax.gather(vec, idx, _dnums, slice_sizes=(1,),
                          mode=lax.GatherScatterMode.PROMISE_IN_BOUNDS)

    def _base(i):
        return (wid + _NW * i) * _CB

    def _idx_copies(i, ri):
        b = _base(i)
        return (pltpu.make_async_copy(dst_hbm.at[pl.ds(b, _CB)], dst_v[ri],
                                      sem_idx[ri]),
                pltpu.make_async_copy(src_hbm.at[pl.ds(b, _CB)], src_v[ri],
                                      sem_idx[ri]))

    def _gather_copies(ri, r2):
        return (pltpu.make_async_copy(p_hbm.at[dst_v[ri]], p_v[r2],
                                      sem_p[r2]),
                pltpu.make_async_copy(q_hbm.at[src_v[ri]], q_v[r2],
                                      sem_q[r2]))

    def _ec_copy(i, r2):
        return pltpu.make_async_copy(ec_hbm.at[pl.ds(_base(i), _CB)],
                                     ec_v[r2], sem_ec[r2])

    def _compute(r2, ri):
        @plsc.parallel_loop(0, _CB // 16, 1, unroll=2)
        def _group_body(g):
            ecg = ec_v[r2][pl.ds(16 * g, 16)]
            for j in range(16):
                e = 16 * g + j
                idxj = jnp.full((16, 1), j, jnp.int32)
                ecv = _vgather(ecg, idxj)
                for f in range(_DH // 16 - 1):
                    pv = p_v[r2][e, pl.ds(16 * f, 16)]
                    qv = q_v[r2][e, pl.ds(16 * f, 16)]
                    h_v[e, pl.ds(16 * f, 16)] = (
                        jnp.maximum(pv + qv, 0.0) * ecv)
                pv = p_v[r2][e, pl.ds(_DH - 16, 16)]
                qv = q_v[r2][e, pl.ds(_DH - 16, 16)]
                h7 = jnp.maximum(pv + qv, 0.0) * ecv
                h_v[e, pl.ds(_DH - 16, 16)] = h7
                perm = _vgather(h7, idx_hi)
                tail = jnp.where(lane_lt8, perm,
                                 jnp.where(lane_eq8, ecv, zero16))
                h_v[e, pl.ds(_DH - 8, 16)] = tail

        pltpu.sync_copy(h_v, s_sh.at[dst_v[ri]], add=True)

    for c in _idx_copies(0, 0):
        c.start()
    for c in _idx_copies(0, 0):
        c.wait()
    for c in _gather_copies(0, 0):
        c.start()
    _ec_copy(0, 0).start()
    for c in _idx_copies(1, 1):
        c.start()
    for c in _idx_copies(2, 2):
        c.start()

    def _slot(i, b):
        r2 = b % 2
        rn2 = (b + 1) % 2
        rn4 = (b + 1) % 4
        rp4 = (b + 3) % 4

        @pl.when(i + 1 < n_i)
        def _():
            for c in _idx_copies(i + 1, rn4):
                c.wait()
            for c in _gather_copies(rn4, rn2):
                c.start()
            _ec_copy(i + 1, rn2).start()

        @pl.when(i + 3 < n_i)
        def _():
            for c in _idx_copies(i + 3, rp4):
                c.start()

        @pl.when(i < n_i)
        def _():
            for c in _gather_copies(b, r2):
                c.wait()
            _ec_copy(i, r2).wait()
            _compute(r2, b)

    def _quad_body(ii, carry):
        for b in range(4):
            _slot(4 * ii + b, b)
        return carry

    lax.fori_loop(0, _NSLOT_QUADS, _quad_body, 0)

    plsc.subcore_barrier()
    for t in range(_ROWS_PER_TILE // _RC):
        pltpu.sync_copy(s_sh.at[pl.ds(row0 + t * _RC, _RC)],
                        out_hbm.at[cid, pl.ds(row0 + t * _RC, _RC)])


@functools.lru_cache(maxsize=1)
def _sc_edge_kernel():
    return pl.kernel(
        _sc_edge_body,
        out_type=jax.ShapeDtypeStruct((_NC, _NPAD, _DS), jnp.float32),
        mesh=plsc.VectorSubcoreMesh(core_axis_name="c", subcore_axis_name="s"),
        compiler_params=pltpu.CompilerParams(use_tc_tiling_on_sc=False),
        scratch_types=[
            [pltpu.VMEM((_CB,), jnp.int32) for _ in range(4)],
            [pltpu.VMEM((_CB,), jnp.int32) for _ in range(4)],
            [pltpu.VMEM((_CB,), jnp.float32) for _ in range(2)],
            [pltpu.VMEM((_CB, _DH), jnp.float32) for _ in range(2)],
            [pltpu.VMEM((_CB, _DH), jnp.float32) for _ in range(2)],
            pltpu.VMEM((_CB, _DS), jnp.float32),
            pltpu.VMEM_SHARED((_NPAD, _DS), jnp.float32),
            [pltpu.SemaphoreType.DMA for _ in range(4)],
            [pltpu.SemaphoreType.DMA for _ in range(2)],
            [pltpu.SemaphoreType.DMA for _ in range(2)],
            [pltpu.SemaphoreType.DMA for _ in range(2)],
        ],
    )


def _post_body(s_ref, x_ref, nc_ref, w2e_ref, u1a_ref, u1b_ref, ub1_ref,
               uw2_ref, ub2_ref, o_ref):
    s = s_ref[0] + s_ref[1]
    message = jnp.dot(s, w2e_ref[...], preferred_element_type=jnp.float32)
    xn = x_ref[...] * nc_ref[...]
    u = (jnp.dot(xn, u1a_ref[...], preferred_element_type=jnp.float32)
         + jnp.dot(message, u1b_ref[...], preferred_element_type=jnp.float32)
         + ub1_ref[...])
    u = jnp.maximum(u, 0.0)
    o_ref[...] = (jnp.dot(u, uw2_ref[...], preferred_element_type=jnp.float32)
                  + ub2_ref[...])


def _post_call(s2, x, nc1, w2ext, u1a, u1b, ub1, uw2, ub2):
    grid = (_N // _BN,)
    full = lambda shape: pl.BlockSpec(shape, lambda i: (0, 0))
    return pl.pallas_call(
        _post_body,
        grid=grid,
        in_specs=[
            pl.BlockSpec((_NC, _BN, _DS), lambda i: (0, i, 0)),
            pl.BlockSpec((_BN, _DIN), lambda i: (i, 0)),
            pl.BlockSpec((_BN, 1), lambda i: (i, 0)),
            full((_DS, _DH)),
            full((_DIN, _DH)),
            full((_DH, _DH)),
            full((1, _DH)),
            full((_DH, _DH)),
            full((1, _DH)),
        ],
        out_specs=pl.BlockSpec((_BN, _DH), lambda i: (i, 0)),
        out_shape=jax.ShapeDtypeStruct((_N, _DH), jnp.float32),
    )(s2, x, nc1, w2ext, u1a, u1b, ub1, uw2, ub2)


def kernel(x, edge_index, node_centrality, edge_centrality, identifiers,
           degrees, msg_W1, msg_b1, msg_W2, msg_b2,
           upd_W1, upd_b1, upd_W2, upd_b2):
    dst = edge_index[1]
    src = edge_index[0]

    p, q = _pre_call(
        x, identifiers,
        msg_W1[0:_DIN], msg_W1[_DIN:2 * _DIN],
        msg_W1[2 * _DIN:2 * _DIN + _DID], msg_W1[2 * _DIN + _DID:],
        msg_b1.reshape(1, _DH),
    )

    s2 = _sc_edge_kernel()(p, q, dst, src, edge_centrality)

    w2ext = jnp.concatenate(
        [msg_W2, msg_b2.reshape(1, _DH),
         jnp.zeros((_DS - _DH - 1, _DH), jnp.float32)], axis=0)

    return _post_call(
        s2, x, node_centrality.reshape(_N, 1), w2ext,
        upd_W1[0:_DIN], upd_W1[_DIN:],
        upd_b1.reshape(1, _DH), upd_W2, upd_b2.reshape(1, _DH))


# speedup vs baseline: 3.4531x; 3.4531x over previous
import functools

import jax
import jax.numpy as jnp
from jax import lax
from jax.experimental import pallas as pl
from jax.experimental.pallas import tpu as pltpu
from jax.experimental.pallas import tpu_sc as plsc

_N = 10000
_E = 320000
_DIN = 128
_DID = 16
_DH = 128
_DS = 136
_CB = 64
_NC = 2
_NS = 16
_NW = _NC * _NS
_NCHUNKS = _E // _CB
_NI_BASE = _NCHUNKS // _NW
_NI_EXTRA = _NCHUNKS - _NI_BASE * _NW
_NSLOT_QUADS = (_NI_BASE + 1 + 3) // 4
_NPAD = 10240
_ROWS_PER_TILE = _NPAD // _NS
_RC = 64


_BN = 1000


def _pre_body(x_ref, id_ref, wxa_ref, wxb_ref, wia_ref, wib_ref, b1_ref,
              p_ref, q_ref):
    xb = x_ref[...]
    idb = id_ref[...]
    p_ref[...] = (jnp.dot(xb, wxa_ref[...], preferred_element_type=jnp.float32)
                  + jnp.dot(idb, wia_ref[...], preferred_element_type=jnp.float32))
    q_ref[...] = (jnp.dot(xb, wxb_ref[...], preferred_element_type=jnp.float32)
                  + jnp.dot(idb, wib_ref[...], preferred_element_type=jnp.float32)
                  + b1_ref[...])


def _pre_call(x, ident, wxa, wxb, wia, wib, b1):
    grid = (_N // _BN,)
    full = lambda shape: pl.BlockSpec(shape, lambda i: (0, 0))
    return pl.pallas_call(
        _pre_body,
        grid=grid,
        in_specs=[
            pl.BlockSpec((_BN, _DIN), lambda i: (i, 0)),
            pl.BlockSpec((_BN, _DID), lambda i: (i, 0)),
            full((_DIN, _DH)),
            full((_DIN, _DH)),
            full((_DID, _DH)),
            full((_DID, _DH)),
            full((1, _DH)),
        ],
        out_specs=[
            pl.BlockSpec((_BN, _DH), lambda i: (i, 0)),
            pl.BlockSpec((_BN, _DH), lambda i: (i, 0)),
        ],
        out_shape=[
            jax.ShapeDtypeStruct((_N, _DH), jnp.float32),
            jax.ShapeDtypeStruct((_N, _DH), jnp.float32),
        ],
    )(x, ident, wxa, wxb, wia, wib, b1)


def _sc_edge_body(p_hbm, q_hbm, dst_hbm, src_hbm, ec_hbm, out_hbm,
                  dst_v, src_v, ec_v, p_v, q_v, h_v, s_sh,
                  sem_idx, sem_p, sem_q, sem_ec):
    cid = lax.axis_index("c")
    sid = lax.axis_index("s")
    wid = sid * _NC + cid
    n_i = _NI_BASE + jnp.where(wid < _NI_EXTRA, 1, 0)

    zero16 = jnp.zeros((16,), jnp.float32)

    def _zero_row(r, carry):
        for f in range(_DS // 16 + 1):
            off = min(16 * f, _DS - 16)
            h_v[r, pl.ds(off, 16)] = zero16
        return carry

    lax.fori_loop(0, _CB, _zero_row, 0)
    row0 = sid * _ROWS_PER_TILE
    for t in range(_ROWS_PER_TILE // _RC):
        pltpu.sync_copy(h_v.at[pl.ds(0, _RC)],
                        s_sh.at[pl.ds(row0 + t * _RC, _RC)])
    plsc.subcore_barrier()

    lane = lax.iota(jnp.int32, 16)
    lane_lt8 = lane < 8
    lane_eq8 = lane == 8
    idx_hi = ((lane % 8) + 8).reshape(16, 1)
    _dnums = lax.GatherDimensionNumbers(
        offset_dims=(), collapsed_slice_dims=(0,), start_index_map=(0,))

    def _vgather(vec, idx):
        return lax.gather(vec, idx, _dnums, slice_sizes=(1,),
                          mode=lax.GatherScatterMode.PROMISE_IN_BOUNDS)

    def _base(i):
        return (wid + _NW * i) * _CB

    def _idx_copies(i, ri):
        b = _base(i)
        return (pltpu.make_async_copy(dst_hbm.at[pl.ds(b, _CB)], dst_v[ri],
                                      sem_idx[ri]),
                pltpu.make_async_copy(src_hbm.at[pl.ds(b, _CB)], src_v[ri],
                                      sem_idx[ri]))

    def _gather_copies(ri, r2):
        return (pltpu.make_async_copy(p_hbm.at[dst_v[ri]], p_v[r2],
                                      sem_p[r2]),
                pltpu.make_async_copy(q_hbm.at[src_v[ri]], q_v[r2],
                                      sem_q[r2]))

    def _ec_copy(i, r2):
        return pltpu.make_async_copy(ec_hbm.at[pl.ds(_base(i), _CB)],
                                     ec_v[r2], sem_ec[r2])

    def _compute(r2, ri):
        def _group_body(g, c2):
            ecg = ec_v[r2][pl.ds(16 * g, 16)]
            for j in range(16):
                e = 16 * g + j
                idxj = jnp.full((16, 1), j, jnp.int32)
                ecv = _vgather(ecg, idxj)
                nf = _DH // 16
                pvs = [p_v[r2][e, pl.ds(16 * f, 16)] for f in range(nf)]
                qvs = [q_v[r2][e, pl.ds(16 * f, 16)] for f in range(nf)]
                hs = [jnp.maximum(pvs[f] + qvs[f], 0.0) * ecv
                      for f in range(nf)]
                for f in range(nf):
                    h_v[e, pl.ds(16 * f, 16)] = hs[f]
                perm = _vgather(hs[nf - 1], idx_hi)
                tail = jnp.where(lane_lt8, perm,
                                 jnp.where(lane_eq8, ecv, zero16))
                h_v[e, pl.ds(_DH - 8, 16)] = tail
            return c2

        lax.fori_loop(0, _CB // 16, _group_body, 0)
        pltpu.sync_copy(h_v, s_sh.at[dst_v[ri]], add=True)

    for c in _idx_copies(0, 0):
        c.start()
    for c in _idx_copies(0, 0):
        c.wait()
    for c in _gather_copies(0, 0):
        c.start()
    _ec_copy(0, 0).start()
    for c in _idx_copies(1, 1):
        c.start()
    for c in _idx_copies(2, 2):
        c.start()

    def _slot(i, b):
        r2 = b % 2
        rn2 = (b + 1) % 2
        rn4 = (b + 1) % 4
        rp4 = (b + 3) % 4

        @pl.when(i + 1 < n_i)
        def _():
            for c in _idx_copies(i + 1, rn4):
                c.wait()
            for c in _gather_copies(rn4, rn2):
                c.start()
            _ec_copy(i + 1, rn2).start()

        @pl.when(i + 3 < n_i)
        def _():
            for c in _idx_copies(i + 3, rp4):
                c.start()

        @pl.when(i < n_i)
        def _():
            for c in _gather_copies(b, r2):
                c.wait()
            _ec_copy(i, r2).wait()
            _compute(r2, b)

    def _quad_body(ii, carry):
        for b in range(4):
            _slot(4 * ii + b, b)
        return carry

    lax.fori_loop(0, _NSLOT_QUADS, _quad_body, 0)

    plsc.subcore_barrier()
    for t in range(_ROWS_PER_TILE // _RC):
        pltpu.sync_copy(s_sh.at[pl.ds(row0 + t * _RC, _RC)],
                        out_hbm.at[cid, pl.ds(row0 + t * _RC, _RC)])


@functools.lru_cache(maxsize=1)
def _sc_edge_kernel():
    return pl.kernel(
        _sc_edge_body,
        out_type=jax.ShapeDtypeStruct((_NC, _NPAD, _DS), jnp.float32),
        mesh=plsc.VectorSubcoreMesh(core_axis_name="c", subcore_axis_name="s"),
        compiler_params=pltpu.CompilerParams(use_tc_tiling_on_sc=False),
        scratch_types=[
            [pltpu.VMEM((_CB,), jnp.int32) for _ in range(4)],
            [pltpu.VMEM((_CB,), jnp.int32) for _ in range(4)],
            [pltpu.VMEM((_CB,), jnp.float32) for _ in range(2)],
            [pltpu.VMEM((_CB, _DH), jnp.float32) for _ in range(2)],
            [pltpu.VMEM((_CB, _DH), jnp.float32) for _ in range(2)],
            pltpu.VMEM((_CB, _DS), jnp.float32),
            pltpu.VMEM_SHARED((_NPAD, _DS), jnp.float32),
            [pltpu.SemaphoreType.DMA for _ in range(4)],
            [pltpu.SemaphoreType.DMA for _ in range(2)],
            [pltpu.SemaphoreType.DMA for _ in range(2)],
            [pltpu.SemaphoreType.DMA for _ in range(2)],
        ],
    )


def _post_body(s_ref, x_ref, nc_ref, w2e_ref, u1a_ref, u1b_ref, ub1_ref,
               uw2_ref, ub2_ref, o_ref):
    s = s_ref[0] + s_ref[1]
    message = jnp.dot(s, w2e_ref[...], preferred_element_type=jnp.float32)
    xn = x_ref[...] * nc_ref[...]
    u = (jnp.dot(xn, u1a_ref[...], preferred_element_type=jnp.float32)
         + jnp.dot(message, u1b_ref[...], preferred_element_type=jnp.float32)
         + ub1_ref[...])
    u = jnp.maximum(u, 0.0)
    o_ref[...] = (jnp.dot(u, uw2_ref[...], preferred_element_type=jnp.float32)
                  + ub2_ref[...])


def _post_call(s2, x, nc1, w2ext, u1a, u1b, ub1, uw2, ub2):
    grid = (_N // _BN,)
    full = lambda shape: pl.BlockSpec(shape, lambda i: (0, 0))
    return pl.pallas_call(
        _post_body,
        grid=grid,
        in_specs=[
            pl.BlockSpec((_NC, _BN, _DS), lambda i: (0, i, 0)),
            pl.BlockSpec((_BN, _DIN), lambda i: (i, 0)),
            pl.BlockSpec((_BN, 1), lambda i: (i, 0)),
            full((_DS, _DH)),
            full((_DIN, _DH)),
            full((_DH, _DH)),
            full((1, _DH)),
            full((_DH, _DH)),
            full((1, _DH)),
        ],
        out_specs=pl.BlockSpec((_BN, _DH), lambda i: (i, 0)),
        out_shape=jax.ShapeDtypeStruct((_N, _DH), jnp.float32),
    )(s2, x, nc1, w2ext, u1a, u1b, ub1, uw2, ub2)


def kernel(x, edge_index, node_centrality, edge_centrality, identifiers,
           degrees, msg_W1, msg_b1, msg_W2, msg_b2,
           upd_W1, upd_b1, upd_W2, upd_b2):
    dst = edge_index[1]
    src = edge_index[0]

    p, q = _pre_call(
        x, identifiers,
        msg_W1[0:_DIN], msg_W1[_DIN:2 * _DIN],
        msg_W1[2 * _DIN:2 * _DIN + _DID], msg_W1[2 * _DIN + _DID:],
        msg_b1.reshape(1, _DH),
    )

    s2 = _sc_edge_kernel()(p, q, dst, src, edge_centrality)

    w2ext = jnp.concatenate(
        [msg_W2, msg_b2.reshape(1, _DH),
         jnp.zeros((_DS - _DH - 1, _DH), jnp.float32)], axis=0)

    return _post_call(
        s2, x, node_centrality.reshape(_N, 1), w2ext,
        upd_W1[0:_DIN], upd_W1[_DIN:],
        upd_b1.reshape(1, _DH), upd_W2, upd_b2.reshape(1, _DH))


# speedup vs baseline: 3.5783x; 1.0363x over previous
import functools

import jax
import jax.numpy as jnp
from jax import lax
from jax.experimental import pallas as pl
from jax.experimental.pallas import tpu as pltpu
from jax.experimental.pallas import tpu_sc as plsc

_N = 10000
_E = 320000
_DIN = 128
_DID = 16
_DH = 128
_DS = 136
_CB = 64
_NC = 2
_NS = 16
_NW = _NC * _NS
_NCHUNKS = _E // _CB
_NI_BASE = _NCHUNKS // _NW
_NI_EXTRA = _NCHUNKS - _NI_BASE * _NW
_NSLOT_QUADS = (_NI_BASE + 1 + 3) // 4
_NPAD = 10240
_ROWS_PER_TILE = _NPAD // _NS
_RC = 64


_BN = 1000


def _pre_body(x_ref, id_ref, wxa_ref, wxb_ref, wia_ref, wib_ref, b1_ref,
              p_ref, q_ref):
    xb = x_ref[...]
    idb = id_ref[...]
    p_ref[...] = (jnp.dot(xb, wxa_ref[...], preferred_element_type=jnp.float32)
                  + jnp.dot(idb, wia_ref[...], preferred_element_type=jnp.float32)
                  ).astype(jnp.bfloat16)
    q_ref[...] = (jnp.dot(xb, wxb_ref[...], preferred_element_type=jnp.float32)
                  + jnp.dot(idb, wib_ref[...], preferred_element_type=jnp.float32)
                  + b1_ref[...]).astype(jnp.bfloat16)


def _pre_call(x, ident, wxa, wxb, wia, wib, b1):
    grid = (_N // _BN,)
    full = lambda shape: pl.BlockSpec(shape, lambda i: (0, 0))
    return pl.pallas_call(
        _pre_body,
        grid=grid,
        in_specs=[
            pl.BlockSpec((_BN, _DIN), lambda i: (i, 0)),
            pl.BlockSpec((_BN, _DID), lambda i: (i, 0)),
            full((_DIN, _DH)),
            full((_DIN, _DH)),
            full((_DID, _DH)),
            full((_DID, _DH)),
            full((1, _DH)),
        ],
        out_specs=[
            pl.BlockSpec((_BN, _DH), lambda i: (i, 0)),
            pl.BlockSpec((_BN, _DH), lambda i: (i, 0)),
        ],
        out_shape=[
            jax.ShapeDtypeStruct((_N, _DH), jnp.bfloat16),
            jax.ShapeDtypeStruct((_N, _DH), jnp.bfloat16),
        ],
    )(x, ident, wxa, wxb, wia, wib, b1)


def _sc_edge_body(p_hbm, q_hbm, dst_hbm, src_hbm, ec_hbm, out_hbm,
                  dst_v, src_v, ec_v, p_v, q_v, h_v, s_sh,
                  sem_idx, sem_p, sem_q, sem_ec):
    cid = lax.axis_index("c")
    sid = lax.axis_index("s")
    wid = sid * _NC + cid
    n_i = _NI_BASE + jnp.where(wid < _NI_EXTRA, 1, 0)

    zero16 = jnp.zeros((16,), jnp.float32)

    def _zero_row(r, carry):
        for f in range(_DS // 16 + 1):
            off = min(16 * f, _DS - 16)
            h_v[r, pl.ds(off, 16)] = zero16
        return carry

    lax.fori_loop(0, _CB, _zero_row, 0)
    row0 = sid * _ROWS_PER_TILE
    for t in range(_ROWS_PER_TILE // _RC):
        pltpu.sync_copy(h_v.at[pl.ds(0, _RC)],
                        s_sh.at[pl.ds(row0 + t * _RC, _RC)])
    plsc.subcore_barrier()

    lane = lax.iota(jnp.int32, 16)
    lane_lt8 = lane < 8
    lane_eq8 = lane == 8
    idx_hi = ((lane % 8) + 8).reshape(16, 1)
    _dnums = lax.GatherDimensionNumbers(
        offset_dims=(), collapsed_slice_dims=(0,), start_index_map=(0,))

    def _vgather(vec, idx):
        return lax.gather(vec, idx, _dnums, slice_sizes=(1,),
                          mode=lax.GatherScatterMode.PROMISE_IN_BOUNDS)

    def _base(i):
        return (wid + _NW * i) * _CB

    def _idx_copies(i, ri):
        b = _base(i)
        return (pltpu.make_async_copy(dst_hbm.at[pl.ds(b, _CB)], dst_v[ri],
                                      sem_idx[ri]),
                pltpu.make_async_copy(src_hbm.at[pl.ds(b, _CB)], src_v[ri],
                                      sem_idx[ri]))

    def _gather_copies(ri, r2):
        return (pltpu.make_async_copy(p_hbm.at[dst_v[ri]], p_v[r2],
                                      sem_p[r2]),
                pltpu.make_async_copy(q_hbm.at[src_v[ri]], q_v[r2],
                                      sem_q[r2]))

    def _ec_copy(i, r2):
        return pltpu.make_async_copy(ec_hbm.at[pl.ds(_base(i), _CB)],
                                     ec_v[r2], sem_ec[r2])

    def _compute(r2, ri):
        def _group_body(g, c2):
            ecg = ec_v[r2][pl.ds(16 * g, 16)]
            for j in range(16):
                e = 16 * g + j
                idxj = jnp.full((16, 1), j, jnp.int32)
                ecv = _vgather(ecg, idxj)
                nb = _DH // 32
                pws = [p_v[r2][e, pl.ds(32 * f, 32)] for f in range(nb)]
                qws = [q_v[r2][e, pl.ds(32 * f, 32)] for f in range(nb)]
                pus = [plsc.unpack(w, format=plsc.PackFormat.INTERLEAVED)
                       for w in pws]
                qus = [plsc.unpack(w, format=plsc.PackFormat.INTERLEAVED)
                       for w in qws]
                pvs = [v for ab in pus for v in ab]
                qvs = [v for ab in qus for v in ab]
                nf = _DH // 16
                hs = [jnp.maximum(pvs[f] + qvs[f], 0.0) * ecv
                      for f in range(nf)]
                for f in range(nf):
                    h_v[e, pl.ds(16 * f, 16)] = hs[f]
                perm = _vgather(hs[nf - 1], idx_hi)
                tail = jnp.where(lane_lt8, perm,
                                 jnp.where(lane_eq8, ecv, zero16))
                h_v[e, pl.ds(_DH - 8, 16)] = tail
            return c2

        lax.fori_loop(0, _CB // 16, _group_body, 0)
        pltpu.sync_copy(h_v, s_sh.at[dst_v[ri]], add=True)

    for c in _idx_copies(0, 0):
        c.start()
    for c in _idx_copies(0, 0):
        c.wait()
    for c in _gather_copies(0, 0):
        c.start()
    _ec_copy(0, 0).start()
    for c in _idx_copies(1, 1):
        c.start()
    for c in _idx_copies(2, 2):
        c.start()

    def _slot(i, b):
        r2 = b % 2
        rn2 = (b + 1) % 2
        rn4 = (b + 1) % 4
        rp4 = (b + 3) % 4

        @pl.when(i + 1 < n_i)
        def _():
            for c in _idx_copies(i + 1, rn4):
                c.wait()
            for c in _gather_copies(rn4, rn2):
                c.start()
            _ec_copy(i + 1, rn2).start()

        @pl.when(i + 3 < n_i)
        def _():
            for c in _idx_copies(i + 3, rp4):
                c.start()

        @pl.when(i < n_i)
        def _():
            for c in _gather_copies(b, r2):
                c.wait()
            _ec_copy(i, r2).wait()
            _compute(r2, b)

    def _quad_body(ii, carry):
        for b in range(4):
            _slot(4 * ii + b, b)
        return carry

    lax.fori_loop(0, _NSLOT_QUADS, _quad_body, 0)

    plsc.subcore_barrier()
    for t in range(_ROWS_PER_TILE // _RC):
        pltpu.sync_copy(s_sh.at[pl.ds(row0 + t * _RC, _RC)],
                        out_hbm.at[cid, pl.ds(row0 + t * _RC, _RC)])


@functools.lru_cache(maxsize=1)
def _sc_edge_kernel():
    return pl.kernel(
        _sc_edge_body,
        out_type=jax.ShapeDtypeStruct((_NC, _NPAD, _DS), jnp.float32),
        mesh=plsc.VectorSubcoreMesh(core_axis_name="c", subcore_axis_name="s"),
        compiler_params=pltpu.CompilerParams(use_tc_tiling_on_sc=False,
                                             needs_layout_passes=False),
        scratch_types=[
            [pltpu.VMEM((_CB,), jnp.int32) for _ in range(4)],
            [pltpu.VMEM((_CB,), jnp.int32) for _ in range(4)],
            [pltpu.VMEM((_CB,), jnp.float32) for _ in range(2)],
            [pltpu.VMEM((_CB, _DH), jnp.bfloat16) for _ in range(2)],
            [pltpu.VMEM((_CB, _DH), jnp.bfloat16) for _ in range(2)],
            pltpu.VMEM((_CB, _DS), jnp.float32),
            pltpu.VMEM_SHARED((_NPAD, _DS), jnp.float32),
            [pltpu.SemaphoreType.DMA for _ in range(4)],
            [pltpu.SemaphoreType.DMA for _ in range(2)],
            [pltpu.SemaphoreType.DMA for _ in range(2)],
            [pltpu.SemaphoreType.DMA for _ in range(2)],
        ],
    )


def _post_body(s_ref, x_ref, nc_ref, w2e_ref, u1a_ref, u1b_ref, ub1_ref,
               uw2_ref, ub2_ref, o_ref):
    s = s_ref[0] + s_ref[1]
    message = jnp.dot(s, w2e_ref[...], preferred_element_type=jnp.float32)
    xn = x_ref[...] * nc_ref[...]
    u = (jnp.dot(xn, u1a_ref[...], preferred_element_type=jnp.float32)
         + jnp.dot(message, u1b_ref[...], preferred_element_type=jnp.float32)
         + ub1_ref[...])
    u = jnp.maximum(u, 0.0)
    o_ref[...] = (jnp.dot(u, uw2_ref[...], preferred_element_type=jnp.float32)
                  + ub2_ref[...])


def _post_call(s2, x, nc1, w2ext, u1a, u1b, ub1, uw2, ub2):
    grid = (_N // _BN,)
    full = lambda shape: pl.BlockSpec(shape, lambda i: (0, 0))
    return pl.pallas_call(
        _post_body,
        grid=grid,
        in_specs=[
            pl.BlockSpec((_NC, _BN, _DS), lambda i: (0, i, 0)),
            pl.BlockSpec((_BN, _DIN), lambda i: (i, 0)),
            pl.BlockSpec((_BN, 1), lambda i: (i, 0)),
            full((_DS, _DH)),
            full((_DIN, _DH)),
            full((_DH, _DH)),
            full((1, _DH)),
            full((_DH, _DH)),
            full((1, _DH)),
        ],
        out_specs=pl.BlockSpec((_BN, _DH), lambda i: (i, 0)),
        out_shape=jax.ShapeDtypeStruct((_N, _DH), jnp.float32),
    )(s2, x, nc1, w2ext, u1a, u1b, ub1, uw2, ub2)


def kernel(x, edge_index, node_centrality, edge_centrality, identifiers,
           degrees, msg_W1, msg_b1, msg_W2, msg_b2,
           upd_W1, upd_b1, upd_W2, upd_b2):
    dst = edge_index[1]
    src = edge_index[0]

    p, q = _pre_call(
        x, identifiers,
        msg_W1[0:_DIN], msg_W1[_DIN:2 * _DIN],
        msg_W1[2 * _DIN:2 * _DIN + _DID], msg_W1[2 * _DIN + _DID:],
        msg_b1.reshape(1, _DH),
    )

    s2 = _sc_edge_kernel()(p, q, dst, src, edge_centrality)

    perm = []
    for f in range(_DH // 32):
        perm += [32 * f + 2 * t for t in range(16)]
        perm += [32 * f + 2 * t + 1 for t in range(16)]
    w2ext = jnp.concatenate(
        [msg_W2[jnp.array(perm, dtype=jnp.int32)], msg_b2.reshape(1, _DH),
         jnp.zeros((_DS - _DH - 1, _DH), jnp.float32)], axis=0)

    return _post_call(
        s2, x, node_centrality.reshape(_N, 1), w2ext,
        upd_W1[0:_DIN], upd_W1[_DIN:],
        upd_b1.reshape(1, _DH), upd_W2, upd_b2.reshape(1, _DH))


# speedup vs baseline: 3.9721x; 1.1101x over previous
import functools

import jax
import jax.numpy as jnp
from jax import lax
from jax.experimental import pallas as pl
from jax.experimental.pallas import tpu as pltpu
from jax.experimental.pallas import tpu_sc as plsc

_N = 10000
_E = 320000
_DIN = 128
_DID = 16
_DH = 128
_DS = 136
_CB = 64
_NC = 2
_NS = 16
_NW = _NC * _NS
_NCHUNKS = _E // _CB
_NI_BASE = _NCHUNKS // _NW
_NI_EXTRA = _NCHUNKS - _NI_BASE * _NW
_NSLOT_QUADS = (_NI_BASE + 1 + 3) // 4
_NPAD = 10240
_ROWS_PER_TILE = _NPAD // _NS
_RC = 64


_BN = 1000


def _pre_body(x_ref, id_ref, wxa_ref, wxb_ref, wia_ref, wib_ref, b1_ref,
              p_ref, q_ref):
    xb = x_ref[...]
    idb = id_ref[...]
    p_ref[...] = (jnp.dot(xb, wxa_ref[...], preferred_element_type=jnp.float32)
                  + jnp.dot(idb, wia_ref[...], preferred_element_type=jnp.float32)
                  ).astype(jnp.bfloat16)
    q_ref[...] = (jnp.dot(xb, wxb_ref[...], preferred_element_type=jnp.float32)
                  + jnp.dot(idb, wib_ref[...], preferred_element_type=jnp.float32)
                  + b1_ref[...]).astype(jnp.bfloat16)


def _pre_call(x, ident, wxa, wxb, wia, wib, b1):
    grid = (_N // _BN,)
    full = lambda shape: pl.BlockSpec(shape, lambda i: (0, 0))
    return pl.pallas_call(
        _pre_body,
        grid=grid,
        in_specs=[
            pl.BlockSpec((_BN, _DIN), lambda i: (i, 0)),
            pl.BlockSpec((_BN, _DID), lambda i: (i, 0)),
            full((_DIN, _DH)),
            full((_DIN, _DH)),
            full((_DID, _DH)),
            full((_DID, _DH)),
            full((1, _DH)),
        ],
        out_specs=[
            pl.BlockSpec((_BN, _DH), lambda i: (i, 0)),
            pl.BlockSpec((_BN, _DH), lambda i: (i, 0)),
        ],
        out_shape=[
            jax.ShapeDtypeStruct((_N, _DH), jnp.bfloat16),
            jax.ShapeDtypeStruct((_N, _DH), jnp.bfloat16),
        ],
    )(x, ident, wxa, wxb, wia, wib, b1)


def _sc_edge_body(p_hbm, q_hbm, dst_hbm, src_hbm, ec_hbm, out_hbm,
                  dst_v, src_v, ec_v, p_v, q_v, h_v, s_sh,
                  sem_idx, sem_p, sem_q, sem_ec, sem_sc):
    cid = lax.axis_index("c")
    sid = lax.axis_index("s")
    wid = sid * _NC + cid
    n_i = _NI_BASE + jnp.where(wid < _NI_EXTRA, 1, 0)

    zero16 = jnp.zeros((16,), jnp.float32)

    def _zero_row(r, carry):
        for f in range(_DS // 16 + 1):
            off = min(16 * f, _DS - 16)
            h_v[0][r, pl.ds(off, 16)] = zero16
            h_v[1][r, pl.ds(off, 16)] = zero16
        return carry

    lax.fori_loop(0, _CB, _zero_row, 0)
    row0 = sid * _ROWS_PER_TILE
    for t in range(_ROWS_PER_TILE // _RC):
        pltpu.sync_copy(h_v[0].at[pl.ds(0, _RC)],
                        s_sh.at[pl.ds(row0 + t * _RC, _RC)])
    plsc.subcore_barrier()

    lane = lax.iota(jnp.int32, 16)
    lane_lt8 = lane < 8
    lane_eq8 = lane == 8
    idx_hi = ((lane % 8) + 8).reshape(16, 1)
    _dnums = lax.GatherDimensionNumbers(
        offset_dims=(), collapsed_slice_dims=(0,), start_index_map=(0,))

    def _vgather(vec, idx):
        return lax.gather(vec, idx, _dnums, slice_sizes=(1,),
                          mode=lax.GatherScatterMode.PROMISE_IN_BOUNDS)

    def _base(i):
        return (wid + _NW * i) * _CB

    def _idx_copies(i, ri):
        b = _base(i)
        return (pltpu.make_async_copy(dst_hbm.at[pl.ds(b, _CB)], dst_v[ri],
                                      sem_idx[ri]),
                pltpu.make_async_copy(src_hbm.at[pl.ds(b, _CB)], src_v[ri],
                                      sem_idx[ri]))

    def _gather_copies(ri, r2):
        return (pltpu.make_async_copy(p_hbm.at[dst_v[ri]], p_v[r2],
                                      sem_p[r2]),
                pltpu.make_async_copy(q_hbm.at[src_v[ri]], q_v[r2],
                                      sem_q[r2]))

    def _ec_copy(i, r2):
        return pltpu.make_async_copy(ec_hbm.at[pl.ds(_base(i), _CB)],
                                     ec_v[r2], sem_ec[r2])

    def _scatter_copy(r2, ri):
        return pltpu.make_async_copy(h_v[r2], s_sh.at[dst_v[ri]],
                                     sem_sc[r2])

    def _compute(r2, ri):
        def _group_body(g, c2):
            ecg = ec_v[r2][pl.ds(16 * g, 16)]
            for j in range(16):
                e = 16 * g + j
                idxj = jnp.full((16, 1), j, jnp.int32)
                ecv = _vgather(ecg, idxj)
                nb = _DH // 32
                pws = [p_v[r2][e, pl.ds(32 * f, 32)] for f in range(nb)]
                qws = [q_v[r2][e, pl.ds(32 * f, 32)] for f in range(nb)]
                pus = [plsc.unpack(w, format=plsc.PackFormat.INTERLEAVED)
                       for w in pws]
                qus = [plsc.unpack(w, format=plsc.PackFormat.INTERLEAVED)
                       for w in qws]
                pvs = [v for ab in pus for v in ab]
                qvs = [v for ab in qus for v in ab]
                nf = _DH // 16
                hs = [jnp.maximum(pvs[f] + qvs[f], 0.0) * ecv
                      for f in range(nf)]
                for f in range(nf):
                    h_v[r2][e, pl.ds(16 * f, 16)] = hs[f]
                perm = _vgather(hs[nf - 1], idx_hi)
                tail = jnp.where(lane_lt8, perm,
                                 jnp.where(lane_eq8, ecv, zero16))
                h_v[r2][e, pl.ds(_DH - 8, 16)] = tail
            return c2

        lax.fori_loop(0, _CB // 16, _group_body, 0)

    for c in _idx_copies(0, 0):
        c.start()
    for c in _idx_copies(0, 0):
        c.wait()
    for c in _gather_copies(0, 0):
        c.start()
    _ec_copy(0, 0).start()
    for c in _idx_copies(1, 1):
        c.start()

    def _slot(i, b):
        r2 = b % 2
        rn2 = (b + 1) % 2
        rn4 = (b + 1) % 4
        rf4 = (b + 2) % 4

        @pl.when(i + 1 < n_i)
        def _():
            for c in _idx_copies(i + 1, rn4):
                c.wait()
            for c in _gather_copies(rn4, rn2):
                c.start()
            _ec_copy(i + 1, rn2).start()

        @pl.when(jnp.logical_and(i >= 2, i - 2 < n_i))
        def _():
            _scatter_copy(r2, rf4).wait()

        @pl.when(i + 2 < n_i)
        def _():
            for c in _idx_copies(i + 2, rf4):
                c.start()

        @pl.when(i < n_i)
        def _():
            for c in _gather_copies(b, r2):
                c.wait()
            _ec_copy(i, r2).wait()
            _compute(r2, b)
            _scatter_copy(r2, b).start(add=True)

    def _quad_body(ii, carry):
        for b in range(4):
            _slot(4 * ii + b, b)
        return carry

    lax.fori_loop(0, _NSLOT_QUADS, _quad_body, 0)

    plsc.subcore_barrier()
    for t in range(_ROWS_PER_TILE // _RC):
        pltpu.sync_copy(s_sh.at[pl.ds(row0 + t * _RC, _RC)],
                        out_hbm.at[cid, pl.ds(row0 + t * _RC, _RC)])


@functools.lru_cache(maxsize=1)
def _sc_edge_kernel():
    return pl.kernel(
        _sc_edge_body,
        out_type=jax.ShapeDtypeStruct((_NC, _NPAD, _DS), jnp.float32),
        mesh=plsc.VectorSubcoreMesh(core_axis_name="c", subcore_axis_name="s"),
        compiler_params=pltpu.CompilerParams(use_tc_tiling_on_sc=False,
                                             needs_layout_passes=False),
        scratch_types=[
            [pltpu.VMEM((_CB,), jnp.int32) for _ in range(4)],
            [pltpu.VMEM((_CB,), jnp.int32) for _ in range(4)],
            [pltpu.VMEM((_CB,), jnp.float32) for _ in range(2)],
            [pltpu.VMEM((_CB, _DH), jnp.bfloat16) for _ in range(2)],
            [pltpu.VMEM((_CB, _DH), jnp.bfloat16) for _ in range(2)],
            [pltpu.VMEM((_CB, _DS), jnp.float32) for _ in range(2)],
            pltpu.VMEM_SHARED((_NPAD, _DS), jnp.float32),
            [pltpu.SemaphoreType.DMA for _ in range(4)],
            [pltpu.SemaphoreType.DMA for _ in range(2)],
            [pltpu.SemaphoreType.DMA for _ in range(2)],
            [pltpu.SemaphoreType.DMA for _ in range(2)],
            [pltpu.SemaphoreType.DMA for _ in range(2)],
        ],
    )


def _post_body(s_ref, x_ref, nc_ref, w2e_ref, u1a_ref, u1b_ref, ub1_ref,
               uw2_ref, ub2_ref, o_ref):
    s = s_ref[0] + s_ref[1]
    message = jnp.dot(s, w2e_ref[...], preferred_element_type=jnp.float32)
    xn = x_ref[...] * nc_ref[...]
    u = (jnp.dot(xn, u1a_ref[...], preferred_element_type=jnp.float32)
         + jnp.dot(message, u1b_ref[...], preferred_element_type=jnp.float32)
         + ub1_ref[...])
    u = jnp.maximum(u, 0.0)
    o_ref[...] = (jnp.dot(u, uw2_ref[...], preferred_element_type=jnp.float32)
                  + ub2_ref[...])


def _post_call(s2, x, nc1, w2ext, u1a, u1b, ub1, uw2, ub2):
    grid = (_N // _BN,)
    full = lambda shape: pl.BlockSpec(shape, lambda i: (0, 0))
    return pl.pallas_call(
        _post_body,
        grid=grid,
        in_specs=[
            pl.BlockSpec((_NC, _BN, _DS), lambda i: (0, i, 0)),
            pl.BlockSpec((_BN, _DIN), lambda i: (i, 0)),
            pl.BlockSpec((_BN, 1), lambda i: (i, 0)),
            full((_DS, _DH)),
            full((_DIN, _DH)),
            full((_DH, _DH)),
            full((1, _DH)),
            full((_DH, _DH)),
            full((1, _DH)),
        ],
        out_specs=pl.BlockSpec((_BN, _DH), lambda i: (i, 0)),
        out_shape=jax.ShapeDtypeStruct((_N, _DH), jnp.float32),
    )(s2, x, nc1, w2ext, u1a, u1b, ub1, uw2, ub2)


def kernel(x, edge_index, node_centrality, edge_centrality, identifiers,
           degrees, msg_W1, msg_b1, msg_W2, msg_b2,
           upd_W1, upd_b1, upd_W2, upd_b2):
    dst = edge_index[1]
    src = edge_index[0]

    p, q = _pre_call(
        x, identifiers,
        msg_W1[0:_DIN], msg_W1[_DIN:2 * _DIN],
        msg_W1[2 * _DIN:2 * _DIN + _DID], msg_W1[2 * _DIN + _DID:],
        msg_b1.reshape(1, _DH),
    )

    s2 = _sc_edge_kernel()(p, q, dst, src, edge_centrality)

    perm = []
    for f in range(_DH // 32):
        perm += [32 * f + 2 * t for t in range(16)]
        perm += [32 * f + 2 * t + 1 for t in range(16)]
    w2ext = jnp.concatenate(
        [msg_W2[jnp.array(perm, dtype=jnp.int32)], msg_b2.reshape(1, _DH),
         jnp.zeros((_DS - _DH - 1, _DH), jnp.float32)], axis=0)

    return _post_call(
        s2, x, node_centrality.reshape(_N, 1), w2ext,
        upd_W1[0:_DIN], upd_W1[_DIN:],
        upd_b1.reshape(1, _DH), upd_W2, upd_b2.reshape(1, _DH))


# speedup vs baseline: 4.1371x; 1.0415x over previous
import functools

import jax
import jax.numpy as jnp
from jax import lax
from jax.experimental import pallas as pl
from jax.experimental.pallas import tpu as pltpu
from jax.experimental.pallas import tpu_sc as plsc

_N = 10000
_E = 320000
_DIN = 128
_DID = 16
_DH = 128
_DS = 136
_CB = 80
_NC = 2
_NS = 16
_NW = _NC * _NS
_NCHUNKS = _E // _CB
_NI_BASE = _NCHUNKS // _NW
_NI_EXTRA = _NCHUNKS - _NI_BASE * _NW
_NSLOT_QUADS = (_NI_BASE + 1 + 3) // 4
_NPAD = 10240
_ROWS_PER_TILE = _NPAD // _NS
_RC = 64


_BN = 1000


def _pre_body(x_ref, id_ref, wxa_ref, wxb_ref, wia_ref, wib_ref, b1_ref,
              p_ref, q_ref):
    xb = x_ref[...]
    idb = id_ref[...]
    p_ref[...] = (jnp.dot(xb, wxa_ref[...], preferred_element_type=jnp.float32)
                  + jnp.dot(idb, wia_ref[...], preferred_element_type=jnp.float32)
                  ).astype(jnp.bfloat16)
    q_ref[...] = (jnp.dot(xb, wxb_ref[...], preferred_element_type=jnp.float32)
                  + jnp.dot(idb, wib_ref[...], preferred_element_type=jnp.float32)
                  + b1_ref[...]).astype(jnp.bfloat16)


def _pre_call(x, ident, wxa, wxb, wia, wib, b1):
    grid = (_N // _BN,)
    full = lambda shape: pl.BlockSpec(shape, lambda i: (0, 0))
    return pl.pallas_call(
        _pre_body,
        grid=grid,
        in_specs=[
            pl.BlockSpec((_BN, _DIN), lambda i: (i, 0)),
            pl.BlockSpec((_BN, _DID), lambda i: (i, 0)),
            full((_DIN, _DH)),
            full((_DIN, _DH)),
            full((_DID, _DH)),
            full((_DID, _DH)),
            full((1, _DH)),
        ],
        out_specs=[
            pl.BlockSpec((_BN, _DH), lambda i: (i, 0)),
            pl.BlockSpec((_BN, _DH), lambda i: (i, 0)),
        ],
        out_shape=[
            jax.ShapeDtypeStruct((_N, _DH), jnp.bfloat16),
            jax.ShapeDtypeStruct((_N, _DH), jnp.bfloat16),
        ],
    )(x, ident, wxa, wxb, wia, wib, b1)


def _sc_edge_body(p_hbm, q_hbm, dst_hbm, src_hbm, ec_hbm, out_hbm,
                  dst_v, src_v, ec_v, p_v, q_v, h_v, s_sh,
                  sem_idx, sem_p, sem_q, sem_ec, sem_sc):
    cid = lax.axis_index("c")
    sid = lax.axis_index("s")
    wid = sid * _NC + cid
    n_i = _NI_BASE + jnp.where(wid < _NI_EXTRA, 1, 0)

    zero16 = jnp.zeros((16,), jnp.float32)

    def _zero_row(r, carry):
        for f in range(_DS // 16 + 1):
            off = min(16 * f, _DS - 16)
            h_v[0][r, pl.ds(off, 16)] = zero16
            h_v[1][r, pl.ds(off, 16)] = zero16
        return carry

    lax.fori_loop(0, _CB, _zero_row, 0)
    row0 = sid * _ROWS_PER_TILE
    for t in range(_ROWS_PER_TILE // _RC):
        pltpu.sync_copy(h_v[0].at[pl.ds(0, _RC)],
                        s_sh.at[pl.ds(row0 + t * _RC, _RC)])
    plsc.subcore_barrier()

    lane = lax.iota(jnp.int32, 16)
    lane_lt8 = lane < 8
    lane_eq8 = lane == 8
    idx_hi = ((lane % 8) + 8).reshape(16, 1)
    _dnums = lax.GatherDimensionNumbers(
        offset_dims=(), collapsed_slice_dims=(0,), start_index_map=(0,))

    def _vgather(vec, idx):
        return lax.gather(vec, idx, _dnums, slice_sizes=(1,),
                          mode=lax.GatherScatterMode.PROMISE_IN_BOUNDS)

    def _base(i):
        return (wid + _NW * i) * _CB

    def _idx_copies(i, ri):
        b = _base(i)
        return (pltpu.make_async_copy(dst_hbm.at[pl.ds(b, _CB)], dst_v[ri],
                                      sem_idx[ri]),
                pltpu.make_async_copy(src_hbm.at[pl.ds(b, _CB)], src_v[ri],
                                      sem_idx[ri]))

    def _gather_copies(ri, r2):
        return (pltpu.make_async_copy(p_hbm.at[dst_v[ri]], p_v[r2],
                                      sem_p[r2]),
                pltpu.make_async_copy(q_hbm.at[src_v[ri]], q_v[r2],
                                      sem_q[r2]))

    def _ec_copy(i, r2):
        return pltpu.make_async_copy(ec_hbm.at[pl.ds(_base(i), _CB)],
                                     ec_v[r2], sem_ec[r2])

    def _scatter_copy(r2, ri):
        return pltpu.make_async_copy(h_v[r2], s_sh.at[dst_v[ri]],
                                     sem_sc[r2])

    def _compute(r2, ri):
        def _group_body(g, c2):
            ecg = ec_v[r2][pl.ds(16 * g, 16)]
            for j in range(16):
                e = 16 * g + j
                idxj = jnp.full((16, 1), j, jnp.int32)
                ecv = _vgather(ecg, idxj)
                nb = _DH // 32
                pws = [p_v[r2][e, pl.ds(32 * f, 32)] for f in range(nb)]
                qws = [q_v[r2][e, pl.ds(32 * f, 32)] for f in range(nb)]
                pus = [plsc.unpack(w, format=plsc.PackFormat.INTERLEAVED)
                       for w in pws]
                qus = [plsc.unpack(w, format=plsc.PackFormat.INTERLEAVED)
                       for w in qws]
                pvs = [v for ab in pus for v in ab]
                qvs = [v for ab in qus for v in ab]
                nf = _DH // 16
                hs = [jnp.maximum(pvs[f] + qvs[f], 0.0) * ecv
                      for f in range(nf)]
                for f in range(nf):
                    h_v[r2][e, pl.ds(16 * f, 16)] = hs[f]
                perm = _vgather(hs[nf - 1], idx_hi)
                tail = jnp.where(lane_lt8, perm,
                                 jnp.where(lane_eq8, ecv, zero16))
                h_v[r2][e, pl.ds(_DH - 8, 16)] = tail
            return c2

        lax.fori_loop(0, _CB // 16, _group_body, 0)

    for c in _idx_copies(0, 0):
        c.start()
    for c in _idx_copies(0, 0):
        c.wait()
    for c in _gather_copies(0, 0):
        c.start()
    _ec_copy(0, 0).start()
    for c in _idx_copies(1, 1):
        c.start()

    def _slot(i, b):
        r2 = b % 2
        rn2 = (b + 1) % 2
        rn4 = (b + 1) % 4
        rf4 = (b + 2) % 4

        @pl.when(i + 1 < n_i)
        def _():
            for c in _idx_copies(i + 1, rn4):
                c.wait()
            for c in _gather_copies(rn4, rn2):
                c.start()
            _ec_copy(i + 1, rn2).start()

        @pl.when(jnp.logical_and(i >= 2, i - 2 < n_i))
        def _():
            _scatter_copy(r2, rf4).wait()

        @pl.when(i + 2 < n_i)
        def _():
            for c in _idx_copies(i + 2, rf4):
                c.start()

        @pl.when(i < n_i)
        def _():
            for c in _gather_copies(b, r2):
                c.wait()
            _ec_copy(i, r2).wait()
            _compute(r2, b)
            _scatter_copy(r2, b).start(add=True)

    def _quad_body(ii, carry):
        for b in range(4):
            _slot(4 * ii + b, b)
        return carry

    lax.fori_loop(0, _NSLOT_QUADS, _quad_body, 0)

    plsc.subcore_barrier()
    for t in range(_ROWS_PER_TILE // _RC):
        pltpu.sync_copy(s_sh.at[pl.ds(row0 + t * _RC, _RC)],
                        out_hbm.at[cid, pl.ds(row0 + t * _RC, _RC)])


@functools.lru_cache(maxsize=1)
def _sc_edge_kernel():
    return pl.kernel(
        _sc_edge_body,
        out_type=jax.ShapeDtypeStruct((_NC, _NPAD, _DS), jnp.float32),
        mesh=plsc.VectorSubcoreMesh(core_axis_name="c", subcore_axis_name="s"),
        compiler_params=pltpu.CompilerParams(use_tc_tiling_on_sc=False,
                                             needs_layout_passes=False),
        scratch_types=[
            [pltpu.VMEM((_CB,), jnp.int32) for _ in range(4)],
            [pltpu.VMEM((_CB,), jnp.int32) for _ in range(4)],
            [pltpu.VMEM((_CB,), jnp.float32) for _ in range(2)],
            [pltpu.VMEM((_CB, _DH), jnp.bfloat16) for _ in range(2)],
            [pltpu.VMEM((_CB, _DH), jnp.bfloat16) for _ in range(2)],
            [pltpu.VMEM((_CB, _DS), jnp.float32) for _ in range(2)],
            pltpu.VMEM_SHARED((_NPAD, _DS), jnp.float32),
            [pltpu.SemaphoreType.DMA for _ in range(4)],
            [pltpu.SemaphoreType.DMA for _ in range(2)],
            [pltpu.SemaphoreType.DMA for _ in range(2)],
            [pltpu.SemaphoreType.DMA for _ in range(2)],
            [pltpu.SemaphoreType.DMA for _ in range(2)],
        ],
    )


def _post_body(s_ref, x_ref, nc_ref, w2e_ref, u1a_ref, u1b_ref, ub1_ref,
               uw2_ref, ub2_ref, o_ref):
    s = s_ref[0] + s_ref[1]
    message = jnp.dot(s, w2e_ref[...], preferred_element_type=jnp.float32)
    xn = x_ref[...] * nc_ref[...]
    u = (jnp.dot(xn, u1a_ref[...], preferred_element_type=jnp.float32)
         + jnp.dot(message, u1b_ref[...], preferred_element_type=jnp.float32)
         + ub1_ref[...])
    u = jnp.maximum(u, 0.0)
    o_ref[...] = (jnp.dot(u, uw2_ref[...], preferred_element_type=jnp.float32)
                  + ub2_ref[...])


def _post_call(s2, x, nc1, w2ext, u1a, u1b, ub1, uw2, ub2):
    grid = (_N // _BN,)
    full = lambda shape: pl.BlockSpec(shape, lambda i: (0, 0))
    return pl.pallas_call(
        _post_body,
        grid=grid,
        in_specs=[
            pl.BlockSpec((_NC, _BN, _DS), lambda i: (0, i, 0)),
            pl.BlockSpec((_BN, _DIN), lambda i: (i, 0)),
            pl.BlockSpec((_BN, 1), lambda i: (i, 0)),
            full((_DS, _DH)),
            full((_DIN, _DH)),
            full((_DH, _DH)),
            full((1, _DH)),
            full((_DH, _DH)),
            full((1, _DH)),
        ],
        out_specs=pl.BlockSpec((_BN, _DH), lambda i: (i, 0)),
        out_shape=jax.ShapeDtypeStruct((_N, _DH), jnp.float32),
    )(s2, x, nc1, w2ext, u1a, u1b, ub1, uw2, ub2)


def kernel(x, edge_index, node_centrality, edge_centrality, identifiers,
           degrees, msg_W1, msg_b1, msg_W2, msg_b2,
           upd_W1, upd_b1, upd_W2, upd_b2):
    dst = edge_index[1]
    src = edge_index[0]

    p, q = _pre_call(
        x, identifiers,
        msg_W1[0:_DIN], msg_W1[_DIN:2 * _DIN],
        msg_W1[2 * _DIN:2 * _DIN + _DID], msg_W1[2 * _DIN + _DID:],
        msg_b1.reshape(1, _DH),
    )

    s2 = _sc_edge_kernel()(p, q, dst, src, edge_centrality)

    perm = []
    for f in range(_DH // 32):
        perm += [32 * f + 2 * t for t in range(16)]
        perm += [32 * f + 2 * t + 1 for t in range(16)]
    w2ext = jnp.concatenate(
        [msg_W2[jnp.array(perm, dtype=jnp.int32)], msg_b2.reshape(1, _DH),
         jnp.zeros((_DS - _DH - 1, _DH), jnp.float32)], axis=0)

    return _post_call(
        s2, x, node_centrality.reshape(_N, 1), w2ext,
        upd_W1[0:_DIN], upd_W1[_DIN:],
        upd_b1.reshape(1, _DH), upd_W2, upd_b2.reshape(1, _DH))
